# baseline (speedup 1.0000x reference)
import functools

import jax
import jax.numpy as jnp
from jax import lax
from jax.experimental import pallas as pl
from jax.experimental.pallas import tpu as pltpu
from jax.experimental.pallas import tpu_sc as plsc

E = 8
H = 2048
F = 1408
SF = 5632
T = 4096

_RBT = 512
_BLK = 256
_NBLK = T * 2 // _BLK + (E - 1)
_P = _NBLK * _BLK
_SBT = 1024
_SFT = 512

_NW = 32
_TPW = T // _NW
_CH = 32
_NCH = _TPW // _CH


def _dot_t(a, b):
    return jax.lax.dot_general(a, b, (((1,), (1,)), ((), ())),
                               preferred_element_type=jnp.float32)


def _router_body(x_ref, gw_ref, logits_ref, ti_ref, tw_ref):
    x = x_ref[...]
    logits = _dot_t(x, gw_ref[...])
    logits_ref[...] = logits
    m = jnp.max(logits, axis=-1, keepdims=True)
    p = jnp.exp(logits - m)
    p = p / jnp.sum(p, axis=-1, keepdims=True)
    iota = jax.lax.broadcasted_iota(jnp.int32, logits.shape, 1)
    i1 = jnp.argmax(logits, axis=-1)[:, None]
    oh1 = iota == i1
    i2 = jnp.argmax(jnp.where(oh1, -1e30, logits), axis=-1)[:, None]
    oh2 = iota == i2
    w1 = jnp.sum(jnp.where(oh1, p, 0.0), axis=-1, keepdims=True)
    w2 = jnp.sum(jnp.where(oh2, p, 0.0), axis=-1, keepdims=True)
    s = w1 + w2
    ti_ref[:, 0:1] = i1
    ti_ref[:, 1:2] = i2
    tw_ref[:, 0:1] = w1 / s
    tw_ref[:, 1:2] = w2 / s


_DCH = 512
_DNC = T // _DCH


def _dot_f32(a, b):
    return jax.lax.dot_general(a, b, (((1,), (0,)), ((), ())),
                               preferred_element_type=jnp.float32)


def _strict_lt(n):
    rio = jax.lax.broadcasted_iota(jnp.int32, (n, n), 0)
    cio = jax.lax.broadcasted_iota(jnp.int32, (n, n), 1)
    return (cio < rio).astype(jnp.float32)


def _dispatch_body(ti_ref, slot_ref, be_ref):
    ti = ti_ref[...]
    iota = jax.lax.broadcasted_iota(jnp.int32, (T, E), 1)
    oh0 = (iota == ti[:, 0:1]).astype(jnp.float32)
    oh1 = (iota == ti[:, 1:2]).astype(jnp.float32)

    lts = _strict_lt(_DCH)
    lt8 = _strict_lt(_DNC)

    def ranks(oh):
        intra, tots = [], []
        for c in range(_DNC):
            blk = oh[c * _DCH:(c + 1) * _DCH, :]
            intra.append(_dot_f32(lts, blk))
            tots.append(jnp.sum(blk, axis=0, keepdims=True))
        tot_mat = jnp.concatenate(tots, axis=0)
        cpref = _dot_f32(lt8, tot_mat)
        total = jnp.sum(tot_mat, axis=0, keepdims=True)
        return intra, cpref, total

    intra0, cpref0, tot0 = ranks(oh0)
    intra1, cpref1, tot1 = ranks(oh1)
    tot = tot0 + tot1
    pc = jnp.floor((tot + (_BLK - 1)) * (1.0 / _BLK)) * float(_BLK)
    rio = jax.lax.broadcasted_iota(jnp.int32, (E, E), 0)
    cio = jax.lax.broadcasted_iota(jnp.int32, (E, E), 1)
    mtri = (rio < cio).astype(jnp.float32)
    poff = _dot_f32(pc, mtri)

    for c in range(_DNC):
        lo, hi = c * _DCH, (c + 1) * _DCH
        r0 = intra0[c] + cpref0[c:c + 1, :]
        sl0 = jnp.sum(oh0[lo:hi, :] * (poff + r0), axis=1, keepdims=True)
        slot_ref[lo:hi, 0:1] = sl0.astype(jnp.int32)
        r1 = tot0 + intra1[c] + cpref1[c:c + 1, :]
        sl1 = jnp.sum(oh1[lo:hi, :] * (poff + r1), axis=1, keepdims=True)
        slot_ref[lo:hi, 1:2] = sl1.astype(jnp.int32)

    bstart = (jax.lax.broadcasted_iota(jnp.int32, (_NBLK + 1, E), 0)
              * _BLK).astype(jnp.float32)
    ends = poff + pc
    be = jnp.sum((bstart >= ends).astype(jnp.int32), axis=1, keepdims=True)
    nused = jnp.sum(pc, axis=1, keepdims=True) * (1.0 / _BLK)
    is_last = jax.lax.broadcasted_iota(jnp.int32, (_NBLK + 1, 1), 0) == _NBLK
    be_ref[...] = jnp.where(is_last, nused.astype(jnp.int32),
                            jnp.minimum(be, E - 1))


def _sc_mesh():
    return plsc.VectorSubcoreMesh(core_axis_name="c", subcore_axis_name="s")


def _sc_scatter_body(x_hbm, slot_hbm, xs_hbm, idx_v, rows_v, sem):
    wid = lax.axis_index("s") * 2 + lax.axis_index("c")
    for cc in range(_NCH):
        base = wid * _TPW + cc * _CH
        pltpu.sync_copy(x_hbm.at[pl.ds(base, _CH)], rows_v)
        for k in range(2):
            pltpu.sync_copy(slot_hbm.at[wid, k, cc], idx_v)
            pltpu.async_copy(rows_v, xs_hbm.at[idx_v], sem).wait()


def _sc_gather_body(y_hbm, slot_hbm, yg1_hbm, yg2_hbm, idx_v, rows_v, sem):
    wid = lax.axis_index("s") * 2 + lax.axis_index("c")
    for k in range(2):
        out_hbm = yg1_hbm if k == 0 else yg2_hbm
        for cc in range(_NCH):
            base = wid * _TPW + cc * _CH
            pltpu.sync_copy(slot_hbm.at[wid, k, cc], idx_v)
            pltpu.async_copy(y_hbm.at[idx_v], rows_v, sem).wait()
            pltpu.sync_copy(rows_v, out_hbm.at[pl.ds(base, _CH)])


def _sc_scatter(xf, slot4d):
    kern = pl.kernel(
        _sc_scatter_body,
        out_type=jax.ShapeDtypeStruct((_P, H), jnp.float32),
        mesh=_sc_mesh(),
        scratch_types=[
            pltpu.VMEM((_CH,), jnp.int32),
            pltpu.VMEM((_CH, H), jnp.float32),
            pltpu.SemaphoreType.DMA,
        ],
    )
    return kern(xf, slot4d)


def _sc_gather(y, slot4d):
    kern = pl.kernel(
        _sc_gather_body,
        out_type=[
            jax.ShapeDtypeStruct((T, H), jnp.float32),
            jax.ShapeDtypeStruct((T, H), jnp.float32),
        ],
        mesh=_sc_mesh(),
        scratch_types=[
            pltpu.VMEM((_CH,), jnp.int32),
            pltpu.VMEM((_CH, H), jnp.float32),
            pltpu.SemaphoreType.DMA,
        ],
    )
    return kern(y, slot4d)


def _group_body(be_ref, x_ref, gw_ref, uw_ref, dw_ref, y_ref):
    @pl.when(pl.program_id(0) < be_ref[_NBLK])
    def _work():
        x = x_ref[...].astype(jnp.bfloat16)
        g = _dot_t(x, gw_ref[0])
        u = _dot_t(x, uw_ref[0])
        h = (g * jax.nn.sigmoid(g) * u).astype(jnp.bfloat16)
        y_ref[...] = _dot_t(h, dw_ref[0])


def _shared_body(x_ref, gw_ref, uw_ref, dw_ref, sg_ref, out_ref):
    f = pl.program_id(1)
    nf = pl.num_programs(1)
    x = x_ref[...]
    g = _dot_t(x, gw_ref[...])
    u = _dot_t(x, uw_ref[...])
    h = (g * jax.nn.sigmoid(g) * u).astype(jnp.bfloat16)
    contrib = _dot_t(h, dw_ref[...])

    @pl.when(f == 0)
    def _init():
        out_ref[...] = contrib

    @pl.when(f > 0)
    def _acc():
        out_ref[...] += contrib

    @pl.when(f == nf - 1)
    def _fin():
        sg_logit = jnp.sum(x.astype(jnp.float32) * sg_ref[...].astype(jnp.float32),
                           axis=-1, keepdims=True)
        out_ref[...] *= jax.nn.sigmoid(sg_logit)


def _combine_body(tw_ref, yg1_ref, yg2_ref, sha_ref, shb_ref, out_ref):
    t = pl.program_id(0)
    nt = pl.num_programs(0)
    w0 = tw_ref[:, 0:1]
    w1 = tw_ref[:, 1:2]
    moe = w0 * yg1_ref[...] + w1 * yg2_ref[...]

    @pl.when(t < nt // 2)
    def _a():
        out_ref[...] = moe + sha_ref[...]

    @pl.when(t >= nt // 2)
    def _b():
        out_ref[...] = moe + shb_ref[...]


def kernel(hidden_states, gate_w, expert_gate_w, expert_up_w, expert_down_w,
           shared_gate_w, shared_up_w, shared_down_w, shared_expert_gate_w):
    B, S, _ = hidden_states.shape
    xf = hidden_states.reshape(-1, H)
    xb = xf.astype(jnp.bfloat16)

    logits, ti, tw = pl.pallas_call(
        _router_body,
        grid=(T // _RBT,),
        in_specs=[
            pl.BlockSpec((_RBT, H), lambda t: (t, 0)),
            pl.BlockSpec((E, H), lambda t: (0, 0)),
        ],
        out_specs=[
            pl.BlockSpec((_RBT, E), lambda t: (t, 0)),
            pl.BlockSpec((_RBT, 2), lambda t: (t, 0)),
            pl.BlockSpec((_RBT, 2), lambda t: (t, 0)),
        ],
        out_shape=[
            jax.ShapeDtypeStruct((T, E), jnp.float32),
            jax.ShapeDtypeStruct((T, 2), jnp.int32),
            jax.ShapeDtypeStruct((T, 2), jnp.float32),
        ],
    )(xf, gate_w)

    slot, be2 = pl.pallas_call(
        _dispatch_body,
        grid=(1,),
        in_specs=[pl.BlockSpec((T, 2), lambda i: (0, 0))],
        out_specs=[
            pl.BlockSpec((T, 2), lambda i: (0, 0)),
            pl.BlockSpec((_NBLK + 1, 1), lambda i: (0, 0)),
        ],
        out_shape=[
            jax.ShapeDtypeStruct((T, 2), jnp.int32),
            jax.ShapeDtypeStruct((_NBLK + 1, 1), jnp.int32),
        ],
    )(ti)
    be = be2.reshape(_NBLK + 1)
    slot4d = slot.T.reshape(2, _NW, _NCH, _CH).transpose(1, 0, 2, 3)

    xs = _sc_scatter(xf, slot4d)

    sgw = shared_gate_w.astype(jnp.bfloat16)
    suw = shared_up_w.astype(jnp.bfloat16)
    sdw = shared_down_w.astype(jnp.bfloat16)
    seg = shared_expert_gate_w.astype(jnp.bfloat16)

    def _shared_half(x_half):
        return pl.pallas_call(
            _shared_body,
            grid=(T // 2 // _SBT, SF // _SFT),
            in_specs=[
                pl.BlockSpec((_SBT, H), lambda t, f: (t, 0)),
                pl.BlockSpec((_SFT, H), lambda t, f: (f, 0)),
                pl.BlockSpec((_SFT, H), lambda t, f: (f, 0)),
                pl.BlockSpec((H, _SFT), lambda t, f: (0, f)),
                pl.BlockSpec((1, H), lambda t, f: (0, 0)),
            ],
            out_specs=pl.BlockSpec((_SBT, H), lambda t, f: (t, 0)),
            out_shape=jax.ShapeDtypeStruct((T // 2, H), jnp.float32),
            compiler_params=pltpu.CompilerParams(
                dimension_semantics=("arbitrary", "arbitrary")),
        )(x_half, sgw, suw, sdw, seg)

    shared_a = _shared_half(xb[:T // 2])


    egw = expert_gate_w.astype(jnp.bfloat16)
    euw = expert_up_w.astype(jnp.bfloat16)
    edw = expert_down_w.astype(jnp.bfloat16)

    y = pl.pallas_call(
        _group_body,
        grid_spec=pltpu.PrefetchScalarGridSpec(
            num_scalar_prefetch=1,
            grid=(_NBLK,),
            in_specs=[
                pl.BlockSpec((_BLK, H), lambda b, be_ref: (b, 0)),
                pl.BlockSpec((1, F, H), lambda b, be_ref: (be_ref[b], 0, 0)),
                pl.BlockSpec((1, F, H), lambda b, be_ref: (be_ref[b], 0, 0)),
                pl.BlockSpec((1, H, F), lambda b, be_ref: (be_ref[b], 0, 0)),
            ],
            out_specs=pl.BlockSpec((_BLK, H), lambda b, be_ref: (b, 0)),
        ),
        out_shape=jax.ShapeDtypeStruct((_P, H), jnp.float32),
        compiler_params=pltpu.CompilerParams(
            dimension_semantics=("arbitrary",)),
    )(be, xs, egw, euw, edw)

    yg1, yg2 = _sc_gather(y, slot4d)

    shared_b = _shared_half(xb[T // 2:])


    _CBT = 512
    _NHB = T // 2 // _CBT
    out = pl.pallas_call(
        _combine_body,
        grid=(T // _CBT,),
        in_specs=[
            pl.BlockSpec((_CBT, 2), lambda t: (t, 0)),
            pl.BlockSpec((_CBT, H), lambda t: (t, 0)),
            pl.BlockSpec((_CBT, H), lambda t: (t, 0)),
            pl.BlockSpec((_CBT, H), lambda t: (jnp.minimum(t, _NHB - 1), 0)),
            pl.BlockSpec((_CBT, H),
                         lambda t: (jnp.maximum(t - _NHB, 0), 0)),
        ],
        out_specs=pl.BlockSpec((_CBT, H), lambda t: (t, 0)),
        out_shape=jax.ShapeDtypeStruct((T, H), jnp.float32),
    )(tw, yg1, yg2, shared_a, shared_b)

    return out.reshape(B, S, H), logits


# speedup vs baseline: 1.0809x; 1.0809x over previous
import functools

import jax
import jax.numpy as jnp
from jax import lax
from jax.experimental import pallas as pl
from jax.experimental.pallas import tpu as pltpu
from jax.experimental.pallas import tpu_sc as plsc

E = 8
H = 2048
F = 1408
SF = 5632
T = 4096

_RBT = 512
_BLK = 256
_NBLK = T * 2 // _BLK + (E - 1)
_P = _NBLK * _BLK
_SBT = 1024
_SFT = 256

_NW = 32
_TPW = T // _NW
_CH = 32
_NCH = _TPW // _CH


def _dot_t(a, b):
    return jax.lax.dot_general(a, b, (((1,), (1,)), ((), ())),
                               preferred_element_type=jnp.float32)


def _router_body(x_ref, gw_ref, logits_ref, ti_ref, tw_ref, xb_ref):
    x = x_ref[...]
    logits = _dot_t(x, gw_ref[...])
    logits_ref[...] = logits
    m = jnp.max(logits, axis=-1, keepdims=True)
    p = jnp.exp(logits - m)
    p = p / jnp.sum(p, axis=-1, keepdims=True)
    iota = jax.lax.broadcasted_iota(jnp.int32, logits.shape, 1)
    i1 = jnp.argmax(logits, axis=-1)[:, None]
    oh1 = iota == i1
    i2 = jnp.argmax(jnp.where(oh1, -1e30, logits), axis=-1)[:, None]
    oh2 = iota == i2
    w1 = jnp.sum(jnp.where(oh1, p, 0.0), axis=-1, keepdims=True)
    w2 = jnp.sum(jnp.where(oh2, p, 0.0), axis=-1, keepdims=True)
    s = w1 + w2
    ti_ref[:, 0:1] = i1
    ti_ref[:, 1:2] = i2
    tw_ref[:, 0:1] = w1 / s
    tw_ref[:, 1:2] = w2 / s
    xb_ref[...] = x.astype(jnp.bfloat16)


_DCH = 512
_DNC = T // _DCH


def _dot_f32(a, b):
    return jax.lax.dot_general(a, b, (((1,), (0,)), ((), ())),
                               preferred_element_type=jnp.float32)


def _strict_lt(n):
    rio = jax.lax.broadcasted_iota(jnp.int32, (n, n), 0)
    cio = jax.lax.broadcasted_iota(jnp.int32, (n, n), 1)
    return (cio < rio).astype(jnp.float32)


def _dispatch_body(ti_ref, slot_ref, be_ref):
    ti = ti_ref[...]
    iota = jax.lax.broadcasted_iota(jnp.int32, (T, E), 1)
    oh0 = (iota == ti[:, 0:1]).astype(jnp.float32)
    oh1 = (iota == ti[:, 1:2]).astype(jnp.float32)

    lts = _strict_lt(_DCH)
    lt8 = _strict_lt(_DNC)

    def ranks(oh):
        intra, tots = [], []
        for c in range(_DNC):
            blk = oh[c * _DCH:(c + 1) * _DCH, :]
            intra.append(_dot_f32(lts, blk))
            tots.append(jnp.sum(blk, axis=0, keepdims=True))
        tot_mat = jnp.concatenate(tots, axis=0)
        cpref = _dot_f32(lt8, tot_mat)
        total = jnp.sum(tot_mat, axis=0, keepdims=True)
        return intra, cpref, total

    intra0, cpref0, tot0 = ranks(oh0)
    intra1, cpref1, tot1 = ranks(oh1)
    tot = tot0 + tot1
    pc = jnp.floor((tot + (_BLK - 1)) * (1.0 / _BLK)) * float(_BLK)
    rio = jax.lax.broadcasted_iota(jnp.int32, (E, E), 0)
    cio = jax.lax.broadcasted_iota(jnp.int32, (E, E), 1)
    mtri = (rio < cio).astype(jnp.float32)
    poff = _dot_f32(pc, mtri)

    for c in range(_DNC):
        lo, hi = c * _DCH, (c + 1) * _DCH
        r0 = intra0[c] + cpref0[c:c + 1, :]
        sl0 = jnp.sum(oh0[lo:hi, :] * (poff + r0), axis=1, keepdims=True)
        slot_ref[lo:hi, 0:1] = sl0.astype(jnp.int32)
        r1 = tot0 + intra1[c] + cpref1[c:c + 1, :]
        sl1 = jnp.sum(oh1[lo:hi, :] * (poff + r1), axis=1, keepdims=True)
        slot_ref[lo:hi, 1:2] = sl1.astype(jnp.int32)

    bstart = (jax.lax.broadcasted_iota(jnp.int32, (_NBLK + 1, E), 0)
              * _BLK).astype(jnp.float32)
    ends = poff + pc
    be = jnp.sum((bstart >= ends).astype(jnp.int32), axis=1, keepdims=True)
    nused = jnp.sum(pc, axis=1, keepdims=True) * (1.0 / _BLK)
    is_last = jax.lax.broadcasted_iota(jnp.int32, (_NBLK + 1, 1), 0) == _NBLK
    be_ref[...] = jnp.where(is_last, nused.astype(jnp.int32),
                            jnp.minimum(be, E - 1))


def _sc_mesh():
    return plsc.VectorSubcoreMesh(core_axis_name="c", subcore_axis_name="s")


def _sc_scatter_body(x_hbm, slot_hbm, xs_hbm, idx_v, rows_v, sem):
    wid = lax.axis_index("s") * 2 + lax.axis_index("c")
    for cc in range(_NCH):
        base = wid * _TPW + cc * _CH
        pltpu.sync_copy(x_hbm.at[pl.ds(base, _CH)], rows_v)
        for k in range(2):
            pltpu.sync_copy(slot_hbm.at[wid, k, cc], idx_v)
            pltpu.async_copy(rows_v, xs_hbm.at[idx_v], sem).wait()


def _sc_gather_body(y_hbm, slot_hbm, yg1_hbm, yg2_hbm, idx_v, rows_v, sem):
    wid = lax.axis_index("s") * 2 + lax.axis_index("c")
    for k in range(2):
        out_hbm = yg1_hbm if k == 0 else yg2_hbm
        for cc in range(_NCH):
            base = wid * _TPW + cc * _CH
            pltpu.sync_copy(slot_hbm.at[wid, k, cc], idx_v)
            pltpu.async_copy(y_hbm.at[idx_v], rows_v, sem).wait()
            pltpu.sync_copy(rows_v, out_hbm.at[pl.ds(base, _CH)])


def _sc_scatter(xf, slot4d):
    kern = pl.kernel(
        _sc_scatter_body,
        out_type=jax.ShapeDtypeStruct((_P, H), jnp.float32),
        mesh=_sc_mesh(),
        scratch_types=[
            pltpu.VMEM((_CH,), jnp.int32),
            pltpu.VMEM((_CH, H), jnp.float32),
            pltpu.SemaphoreType.DMA,
        ],
    )
    return kern(xf, slot4d)


def _sc_gather(y, slot4d):
    kern = pl.kernel(
        _sc_gather_body,
        out_type=[
            jax.ShapeDtypeStruct((T, H), jnp.float32),
            jax.ShapeDtypeStruct((T, H), jnp.float32),
        ],
        mesh=_sc_mesh(),
        scratch_types=[
            pltpu.VMEM((_CH,), jnp.int32),
            pltpu.VMEM((_CH, H), jnp.float32),
            pltpu.SemaphoreType.DMA,
        ],
    )
    return kern(y, slot4d)


def _group_body(be_ref, x_ref, gw_ref, uw_ref, dw_ref, y_ref):
    @pl.when(pl.program_id(0) < be_ref[_NBLK])
    def _work():
        x = x_ref[...].astype(jnp.bfloat16)
        g = _dot_t(x, gw_ref[0])
        u = _dot_t(x, uw_ref[0])
        h = (g * jax.nn.sigmoid(g) * u).astype(jnp.bfloat16)
        y_ref[...] = _dot_t(h, dw_ref[0])


def _shared_body(x_ref, gw_ref, uw_ref, dw_ref, sg_ref, out_ref):
    f = pl.program_id(1)
    nf = pl.num_programs(1)
    x = x_ref[...]
    g = _dot_t(x, gw_ref[...].astype(jnp.bfloat16))
    u = _dot_t(x, uw_ref[...].astype(jnp.bfloat16))
    h = (g * jax.nn.sigmoid(g) * u).astype(jnp.bfloat16)
    contrib = _dot_t(h, dw_ref[...].astype(jnp.bfloat16))

    @pl.when(f == 0)
    def _init():
        out_ref[...] = contrib

    @pl.when(f > 0)
    def _acc():
        out_ref[...] += contrib

    @pl.when(f == nf - 1)
    def _fin():
        sg_logit = jnp.sum(x.astype(jnp.float32) * sg_ref[...].astype(jnp.float32),
                           axis=-1, keepdims=True)
        out_ref[...] *= jax.nn.sigmoid(sg_logit)


def _combine_body(tw_ref, yg1_ref, yg2_ref, sh_ref, out_ref):
    w0 = tw_ref[:, 0:1]
    w1 = tw_ref[:, 1:2]
    out_ref[...] = w0 * yg1_ref[...] + w1 * yg2_ref[...] + sh_ref[...]


def kernel(hidden_states, gate_w, expert_gate_w, expert_up_w, expert_down_w,
           shared_gate_w, shared_up_w, shared_down_w, shared_expert_gate_w):
    B, S, _ = hidden_states.shape
    xf = hidden_states.reshape(-1, H)

    logits, ti, tw, xb = pl.pallas_call(
        _router_body,
        grid=(T // _RBT,),
        in_specs=[
            pl.BlockSpec((_RBT, H), lambda t: (t, 0)),
            pl.BlockSpec((E, H), lambda t: (0, 0)),
        ],
        out_specs=[
            pl.BlockSpec((_RBT, E), lambda t: (t, 0)),
            pl.BlockSpec((_RBT, 2), lambda t: (t, 0)),
            pl.BlockSpec((_RBT, 2), lambda t: (t, 0)),
            pl.BlockSpec((_RBT, H), lambda t: (t, 0)),
        ],
        out_shape=[
            jax.ShapeDtypeStruct((T, E), jnp.float32),
            jax.ShapeDtypeStruct((T, 2), jnp.int32),
            jax.ShapeDtypeStruct((T, 2), jnp.float32),
            jax.ShapeDtypeStruct((T, H), jnp.bfloat16),
        ],
    )(xf, gate_w)

    slot, be2 = pl.pallas_call(
        _dispatch_body,
        grid=(1,),
        in_specs=[pl.BlockSpec((T, 2), lambda i: (0, 0))],
        out_specs=[
            pl.BlockSpec((T, 2), lambda i: (0, 0)),
            pl.BlockSpec((_NBLK + 1, 1), lambda i: (0, 0)),
        ],
        out_shape=[
            jax.ShapeDtypeStruct((T, 2), jnp.int32),
            jax.ShapeDtypeStruct((_NBLK + 1, 1), jnp.int32),
        ],
    )(ti)
    be = be2.reshape(_NBLK + 1)
    slot4d = slot.T.reshape(2, _NW, _NCH, _CH).transpose(1, 0, 2, 3)

    xs = _sc_scatter(xf, slot4d)

    seg = shared_expert_gate_w.astype(jnp.bfloat16)

    shared_out = pl.pallas_call(
        _shared_body,
        grid=(T // _SBT, SF // _SFT),
        in_specs=[
            pl.BlockSpec((_SBT, H), lambda t, f: (t, 0)),
            pl.BlockSpec((_SFT, H), lambda t, f: (f, 0)),
            pl.BlockSpec((_SFT, H), lambda t, f: (f, 0)),
            pl.BlockSpec((H, _SFT), lambda t, f: (0, f)),
            pl.BlockSpec((1, H), lambda t, f: (0, 0)),
        ],
        out_specs=pl.BlockSpec((_SBT, H), lambda t, f: (t, 0)),
        out_shape=jax.ShapeDtypeStruct((T, H), jnp.float32),
        compiler_params=pltpu.CompilerParams(
            dimension_semantics=("arbitrary", "arbitrary")),
    )(xb, shared_gate_w, shared_up_w, shared_down_w, seg)


    egw = expert_gate_w.astype(jnp.bfloat16)
    euw = expert_up_w.astype(jnp.bfloat16)
    edw = expert_down_w.astype(jnp.bfloat16)

    y = pl.pallas_call(
        _group_body,
        grid_spec=pltpu.PrefetchScalarGridSpec(
            num_scalar_prefetch=1,
            grid=(_NBLK,),
            in_specs=[
                pl.BlockSpec((_BLK, H), lambda b, be_ref: (b, 0)),
                pl.BlockSpec((1, F, H), lambda b, be_ref: (be_ref[b], 0, 0)),
                pl.BlockSpec((1, F, H), lambda b, be_ref: (be_ref[b], 0, 0)),
                pl.BlockSpec((1, H, F), lambda b, be_ref: (be_ref[b], 0, 0)),
            ],
            out_specs=pl.BlockSpec((_BLK, H), lambda b, be_ref: (b, 0)),
        ),
        out_shape=jax.ShapeDtypeStruct((_P, H), jnp.float32),
        compiler_params=pltpu.CompilerParams(
            dimension_semantics=("arbitrary",)),
    )(be, xs, egw, euw, edw)

    yg1, yg2 = _sc_gather(y, slot4d)


    _CBT = 512
    out = pl.pallas_call(
        _combine_body,
        grid=(T // _CBT,),
        in_specs=[
            pl.BlockSpec((_CBT, 2), lambda t: (t, 0)),
            pl.BlockSpec((_CBT, H), lambda t: (t, 0)),
            pl.BlockSpec((_CBT, H), lambda t: (t, 0)),
            pl.BlockSpec((_CBT, H), lambda t: (t, 0)),
        ],
        out_specs=pl.BlockSpec((_CBT, H), lambda t: (t, 0)),
        out_shape=jax.ShapeDtypeStruct((T, H), jnp.float32),
    )(tw, yg1, yg2, shared_out)

    return out.reshape(B, S, H), logits
